# Initial kernel scaffold; baseline (speedup 1.0000x reference)
import jax
import jax.numpy as jnp
from jax.experimental import pallas as pl


def kernel(x, edge_index, batch, W0, b0, W1, b1, W2, b2, Wf1, bf1, Wf2, bf2):
    raise NotImplementedError("write your pallas kernel here")


# speedup vs baseline: 12.5131x; 12.5131x over previous
import functools

import jax
import jax.numpy as jnp
from jax import lax
from jax.experimental import pallas as pl
from jax.experimental.pallas import tpu as pltpu
from jax.experimental.pallas import tpu_sc as plsc

F32 = jnp.float32
I32 = jnp.int32

N = 50000
E = 800000
NODE_IN = 163
H = 64
HH = H // 2
G = 512

NPAD = 51200
EPAD = 819200
EC = 640
E_PER_TILE = EPAD // 16
E_CHUNKS = E_PER_TILE // EC
W_PER_DEG = EPAD // 32
DEG_CHUNKS = W_PER_DEG // EC
NSLICE = NPAD // 16
SENT_NODE = NPAD - 8
PC = 640
P_CHUNKS = NSLICE // PC
GP = 520

_mesh = plsc.VectorSubcoreMesh(core_axis_name="c", subcore_axis_name="s")


@functools.partial(
    pl.kernel,
    out_type=(jax.ShapeDtypeStruct((NPAD,), F32),
              jax.ShapeDtypeStruct((NPAD,), F32)),
    mesh=_mesh,
    compiler_params=pltpu.CompilerParams(use_tc_tiling_on_sc=False),
    scratch_types=[
        pltpu.VMEM((EC,), I32),
        pltpu.VMEM((EC,), F32),
        pltpu.VMEM_SHARED((NPAD,), F32),
    ],
)
def _deg_kernel(dst_hbm, zeros1_hbm, ones_hbm, degA_hbm, degB_hbm,
                didx_v, ones_v, acc_sh):
    cid = lax.axis_index("c")
    sid = lax.axis_index("s")
    pltpu.sync_copy(zeros1_hbm, acc_sh.at[pl.ds(sid * NSLICE, NSLICE)])
    pltpu.sync_copy(ones_hbm, ones_v)
    plsc.subcore_barrier()

    w = cid * 16 + sid

    def body(i, carry):
        base = w * W_PER_DEG + i * EC
        pltpu.sync_copy(dst_hbm.at[pl.ds(base, EC)], didx_v)
        pltpu.sync_copy(ones_v, acc_sh.at[didx_v], add=True)
        return carry

    lax.fori_loop(0, DEG_CHUNKS, body, 0)
    plsc.subcore_barrier()

    sl = pl.ds(sid * NSLICE, NSLICE)

    @pl.when(cid == 0)
    def _():
        pltpu.sync_copy(acc_sh.at[sl], degA_hbm.at[sl])

    @pl.when(cid == 1)
    def _():
        pltpu.sync_copy(acc_sh.at[sl], degB_hbm.at[sl])


@functools.partial(
    pl.kernel,
    out_type=(jax.ShapeDtypeStruct((NPAD, HH), F32),
              jax.ShapeDtypeStruct((NPAD, HH), F32)),
    mesh=_mesh,
    compiler_params=pltpu.CompilerParams(use_tc_tiling_on_sc=False),
    scratch_types=[
        pltpu.VMEM((EC,), I32),
        pltpu.VMEM((EC,), I32),
        pltpu.VMEM((EC, HH), F32),
        pltpu.VMEM_SHARED((NPAD, HH), F32),
        pltpu.SemaphoreType.DMA,
    ],
)
def _edge_kernel(src_hbm, dst_hbm, xsA_hbm, xsB_hbm, zeros2_hbm,
                 outA_hbm, outB_hbm, sidx_v, didx_v, rows_v, acc_sh, sem):
    cid = lax.axis_index("c")
    sid = lax.axis_index("s")
    pltpu.sync_copy(zeros2_hbm, acc_sh.at[pl.ds(sid * NSLICE, NSLICE)])
    plsc.subcore_barrier()

    def run(xs_hbm):
        def body(i, carry):
            base = sid * E_PER_TILE + i * EC
            pltpu.sync_copy(src_hbm.at[pl.ds(base, EC)], sidx_v)
            pltpu.sync_copy(dst_hbm.at[pl.ds(base, EC)], didx_v)
            pltpu.async_copy(xs_hbm.at[sidx_v], rows_v, sem).wait()
            pltpu.sync_copy(rows_v, acc_sh.at[didx_v], add=True)
            return carry
        lax.fori_loop(0, E_CHUNKS, body, 0)

    @pl.when(cid == 0)
    def _():
        run(xsA_hbm)

    @pl.when(cid == 1)
    def _():
        run(xsB_hbm)

    plsc.subcore_barrier()
    sl = pl.ds(sid * NSLICE, NSLICE)

    @pl.when(cid == 0)
    def _():
        pltpu.sync_copy(acc_sh.at[sl], outA_hbm.at[sl])

    @pl.when(cid == 1)
    def _():
        pltpu.sync_copy(acc_sh.at[sl], outB_hbm.at[sl])


@functools.partial(
    pl.kernel,
    out_type=(jax.ShapeDtypeStruct((G, HH), F32),
              jax.ShapeDtypeStruct((G, HH), F32),
              jax.ShapeDtypeStruct((G,), F32)),
    mesh=_mesh,
    compiler_params=pltpu.CompilerParams(use_tc_tiling_on_sc=False),
    scratch_types=[
        pltpu.VMEM((PC,), I32),
        pltpu.VMEM((PC, HH), F32),
        pltpu.VMEM((PC,), F32),
        pltpu.VMEM_SHARED((GP, HH), F32),
        pltpu.VMEM_SHARED((GP,), F32),
    ],
)
def _pool_kernel(hA_hbm, hB_hbm, batch_hbm, zeros2_hbm, zeros1_hbm, ones_hbm,
                 sumsA_hbm, sumsB_hbm, cnt_hbm,
                 bidx_v, rows_v, ones_v, accP_sh, accC_sh):
    cid = lax.axis_index("c")
    sid = lax.axis_index("s")

    @pl.when(sid == 0)
    def _():
        pltpu.sync_copy(zeros2_hbm.at[pl.ds(0, GP)], accP_sh)
        pltpu.sync_copy(zeros1_hbm.at[pl.ds(0, GP)], accC_sh)

    pltpu.sync_copy(ones_hbm, ones_v)
    plsc.subcore_barrier()

    def run(h_hbm, do_cnt):
        def body(i, carry):
            r0 = sid * NSLICE + i * PC
            pltpu.sync_copy(batch_hbm.at[pl.ds(r0, PC)], bidx_v)
            pltpu.sync_copy(h_hbm.at[pl.ds(r0, PC)], rows_v)
            pltpu.sync_copy(rows_v, accP_sh.at[bidx_v], add=True)
            if do_cnt:
                pltpu.sync_copy(ones_v, accC_sh.at[bidx_v], add=True)
            return carry
        lax.fori_loop(0, P_CHUNKS, body, 0)

    @pl.when(cid == 0)
    def _():
        run(hA_hbm, True)

    @pl.when(cid == 1)
    def _():
        run(hB_hbm, False)

    plsc.subcore_barrier()

    @pl.when(sid == 0)
    def _():
        @pl.when(cid == 0)
        def _():
            pltpu.sync_copy(accP_sh.at[pl.ds(0, G)], sumsA_hbm)
            pltpu.sync_copy(accC_sh.at[pl.ds(0, G)], cnt_hbm)

        @pl.when(cid == 1)
        def _():
            pltpu.sync_copy(accP_sh.at[pl.ds(0, G)], sumsB_hbm)


RB = 512
N_BLOCKS = NPAD // RB


def _leaky_tc(v):
    return jnp.where(v >= 0, v, 0.01 * v)


def _t0_body(x_ref, dA_ref, dB_ref, W_ref, xsA_ref, xsB_ref, dv_ref):
    deg = dA_ref[...] + dB_ref[...] + 1.0
    dv = lax.rsqrt(deg)
    xw = jnp.dot(x_ref[...], W_ref[...], preferred_element_type=F32)
    xs = xw * dv
    xsA_ref[...] = xs[:, :HH]
    xsB_ref[...] = xs[:, HH:]
    dv_ref[...] = dv


_t0_call = pl.pallas_call(
    _t0_body,
    grid=(N_BLOCKS,),
    in_specs=[
        pl.BlockSpec((RB, NODE_IN), lambda i: (i, 0)),
        pl.BlockSpec((RB, 1), lambda i: (i, 0)),
        pl.BlockSpec((RB, 1), lambda i: (i, 0)),
        pl.BlockSpec((NODE_IN, H), lambda i: (0, 0)),
    ],
    out_specs=[
        pl.BlockSpec((RB, HH), lambda i: (i, 0)),
        pl.BlockSpec((RB, HH), lambda i: (i, 0)),
        pl.BlockSpec((RB, 1), lambda i: (i, 0)),
    ],
    out_shape=[
        jax.ShapeDtypeStruct((NPAD, HH), F32),
        jax.ShapeDtypeStruct((NPAD, HH), F32),
        jax.ShapeDtypeStruct((NPAD, 1), F32),
    ],
)


def _t12_body(SA_ref, SB_ref, xA_ref, xB_ref, dv_ref, b_ref, W_ref,
              outA_ref, outB_ref):
    dv = dv_ref[...]
    b = b_ref[...]
    hA = _leaky_tc(dv * (SA_ref[...] + xA_ref[...]) + b[:, :HH])
    hB = _leaky_tc(dv * (SB_ref[...] + xB_ref[...]) + b[:, HH:])
    h = jnp.concatenate([hA, hB], axis=1)
    xw = jnp.dot(h, W_ref[...], preferred_element_type=F32)
    xs = xw * dv
    outA_ref[...] = xs[:, :HH]
    outB_ref[...] = xs[:, HH:]


_t12_call = pl.pallas_call(
    _t12_body,
    grid=(N_BLOCKS,),
    in_specs=[
        pl.BlockSpec((RB, HH), lambda i: (i, 0)),
        pl.BlockSpec((RB, HH), lambda i: (i, 0)),
        pl.BlockSpec((RB, HH), lambda i: (i, 0)),
        pl.BlockSpec((RB, HH), lambda i: (i, 0)),
        pl.BlockSpec((RB, 1), lambda i: (i, 0)),
        pl.BlockSpec((1, H), lambda i: (0, 0)),
        pl.BlockSpec((H, H), lambda i: (0, 0)),
    ],
    out_specs=[
        pl.BlockSpec((RB, HH), lambda i: (i, 0)),
        pl.BlockSpec((RB, HH), lambda i: (i, 0)),
    ],
    out_shape=[
        jax.ShapeDtypeStruct((NPAD, HH), F32),
        jax.ShapeDtypeStruct((NPAD, HH), F32),
    ],
)


def _t3_body(SA_ref, SB_ref, xA_ref, xB_ref, dv_ref, b_ref,
             hA_ref, hB_ref):
    dv = dv_ref[...]
    b = b_ref[...]
    hA_ref[...] = _leaky_tc(dv * (SA_ref[...] + xA_ref[...]) + b[:, :HH])
    hB_ref[...] = _leaky_tc(dv * (SB_ref[...] + xB_ref[...]) + b[:, HH:])


_t3_call = pl.pallas_call(
    _t3_body,
    grid=(N_BLOCKS,),
    in_specs=[
        pl.BlockSpec((RB, HH), lambda i: (i, 0)),
        pl.BlockSpec((RB, HH), lambda i: (i, 0)),
        pl.BlockSpec((RB, HH), lambda i: (i, 0)),
        pl.BlockSpec((RB, HH), lambda i: (i, 0)),
        pl.BlockSpec((RB, 1), lambda i: (i, 0)),
        pl.BlockSpec((1, H), lambda i: (0, 0)),
    ],
    out_specs=[
        pl.BlockSpec((RB, HH), lambda i: (i, 0)),
        pl.BlockSpec((RB, HH), lambda i: (i, 0)),
    ],
    out_shape=[
        jax.ShapeDtypeStruct((NPAD, HH), F32),
        jax.ShapeDtypeStruct((NPAD, HH), F32),
    ],
)


def _head_body(sA_ref, sB_ref, cnt_ref, Wf1_ref, bf1_ref, Wf2_ref, bf2_ref,
               out_ref):
    pooled = jnp.concatenate([sA_ref[...], sB_ref[...]], axis=1)
    pooled = pooled / jnp.maximum(cnt_ref[...], 1.0)
    z = _leaky_tc(
        jnp.dot(pooled, Wf1_ref[...], preferred_element_type=F32)
        + bf1_ref[...])
    out_ref[...] = (
        jnp.dot(z, Wf2_ref[...], preferred_element_type=F32) + bf2_ref[...])


_head_call = pl.pallas_call(
    _head_body,
    out_shape=jax.ShapeDtypeStruct((G, 1), F32),
)


def kernel(x, edge_index, batch, W0, b0, W1, b1, W2, b2, Wf1, bf1, Wf2, bf2):
    x_pad = jnp.pad(x, ((0, NPAD - N), (0, 0)))
    sent = jnp.full((EPAD - E,), SENT_NODE, dtype=I32)
    src_pad = jnp.concatenate([edge_index[0], sent])
    dst_pad = jnp.concatenate([edge_index[1], sent])
    batch_pad = jnp.concatenate(
        [batch, jnp.full((NPAD - N,), G, dtype=I32)])
    zeros2 = jnp.zeros((NSLICE, HH), dtype=F32)
    zeros1 = jnp.zeros((NSLICE,), dtype=F32)
    ones_e = jnp.ones((EC,), dtype=F32)
    ones_p = jnp.ones((PC,), dtype=F32)

    degA, degB = _deg_kernel(dst_pad, zeros1, ones_e)
    xsA, xsB, dv = _t0_call(x_pad, degA.reshape(NPAD, 1),
                            degB.reshape(NPAD, 1), W0)
    SA, SB = _edge_kernel(src_pad, dst_pad, xsA, xsB, zeros2)
    xsA, xsB = _t12_call(SA, SB, xsA, xsB, dv, b0.reshape(1, H), W1)
    SA, SB = _edge_kernel(src_pad, dst_pad, xsA, xsB, zeros2)
    xsA, xsB = _t12_call(SA, SB, xsA, xsB, dv, b1.reshape(1, H), W2)
    SA, SB = _edge_kernel(src_pad, dst_pad, xsA, xsB, zeros2)
    hA, hB = _t3_call(SA, SB, xsA, xsB, dv, b2.reshape(1, H))
    sumsA, sumsB, cnt = _pool_kernel(hA, hB, batch_pad, zeros2, zeros1, ones_p)
    out = _head_call(sumsA, sumsB, cnt.reshape(G, 1),
                     Wf1, bf1.reshape(1, H), Wf2, bf2.reshape(1, 1))
    return out.reshape(G)


# speedup vs baseline: 24.4743x; 1.9559x over previous
import functools

import jax
import jax.numpy as jnp
from jax import lax
from jax.experimental import pallas as pl
from jax.experimental.pallas import tpu as pltpu
from jax.experimental.pallas import tpu_sc as plsc

F32 = jnp.float32
I32 = jnp.int32

N = 50000
E = 800000
NODE_IN = 163
H = 64
HH = H // 2
G = 512

NPAD = 50176
EC = 400
E_PER_TILE = E // 16
E_CHUNKS = E_PER_TILE // EC
W_PER_DEG = E // 32
DEG_EC = 1000
DEG_CHUNKS = W_PER_DEG // DEG_EC
NSLICE = NPAD // 16
PC = 784
P_CHUNKS = NSLICE // PC
GP = 520

_mesh = plsc.VectorSubcoreMesh(core_axis_name="c", subcore_axis_name="s")


@functools.partial(
    pl.kernel,
    out_type=(jax.ShapeDtypeStruct((NPAD,), F32),
              jax.ShapeDtypeStruct((NPAD,), F32)),
    mesh=_mesh,
    compiler_params=pltpu.CompilerParams(use_tc_tiling_on_sc=False),
    scratch_types=[
        pltpu.VMEM((DEG_EC,), I32),
        pltpu.VMEM((DEG_EC,), F32),
        pltpu.VMEM_SHARED((NPAD,), F32),
    ],
)
def _deg_kernel(dst_hbm, zeros1_hbm, ones_hbm, degA_hbm, degB_hbm,
                didx_v, ones_v, acc_sh):
    cid = lax.axis_index("c")
    sid = lax.axis_index("s")
    pltpu.sync_copy(zeros1_hbm, acc_sh.at[pl.ds(sid * NSLICE, NSLICE)])
    pltpu.sync_copy(ones_hbm, ones_v)
    plsc.subcore_barrier()

    w = cid * 16 + sid

    def body(i, carry):
        base = w * W_PER_DEG + i * DEG_EC
        pltpu.sync_copy(dst_hbm.at[pl.ds(base, DEG_EC)], didx_v)
        pltpu.sync_copy(ones_v, acc_sh.at[didx_v], add=True)
        return carry

    lax.fori_loop(0, DEG_CHUNKS, body, 0)
    plsc.subcore_barrier()

    sl = pl.ds(sid * NSLICE, NSLICE)

    @pl.when(cid == 0)
    def _():
        pltpu.sync_copy(acc_sh.at[sl], degA_hbm.at[sl])

    @pl.when(cid == 1)
    def _():
        pltpu.sync_copy(acc_sh.at[sl], degB_hbm.at[sl])


@functools.partial(
    pl.kernel,
    out_type=(jax.ShapeDtypeStruct((NPAD, HH), F32),
              jax.ShapeDtypeStruct((NPAD, HH), F32)),
    mesh=_mesh,
    compiler_params=pltpu.CompilerParams(use_tc_tiling_on_sc=False),
    scratch_types=[
        pltpu.VMEM((EC,), I32),
        pltpu.VMEM((EC,), I32),
        pltpu.VMEM((EC,), I32),
        pltpu.VMEM((EC,), I32),
        pltpu.VMEM((EC, HH), F32),
        pltpu.VMEM((EC, HH), F32),
        pltpu.VMEM_SHARED((NPAD, HH), F32),
        pltpu.SemaphoreType.DMA,
        pltpu.SemaphoreType.DMA,
        pltpu.SemaphoreType.DMA,
        pltpu.SemaphoreType.DMA,
    ],
)
def _edge_kernel(src_hbm, dst_hbm, xsA_hbm, xsB_hbm, zeros2_hbm,
                 outA_hbm, outB_hbm,
                 sidx0, didx0, sidx1, didx1, rows0, rows1, acc_sh,
                 isem0, isem1, gsem0, gsem1):
    cid = lax.axis_index("c")
    sid = lax.axis_index("s")
    pltpu.sync_copy(zeros2_hbm, acc_sh.at[pl.ds(sid * NSLICE, NSLICE)])
    plsc.subcore_barrier()

    sidx = (sidx0, sidx1)
    didx = (didx0, didx1)
    rows = (rows0, rows1)
    isem = (isem0, isem1)
    gsem = (gsem0, gsem1)

    def run(xs_hbm):
        def I(c, b):
            base = sid * E_PER_TILE + c * EC
            pltpu.async_copy(src_hbm.at[pl.ds(base, EC)], sidx[b], isem[b])
            pltpu.async_copy(dst_hbm.at[pl.ds(base, EC)], didx[b], isem[b])

        def Iw(c, b):
            @pl.when(c < E_CHUNKS)
            def _():
                I(c, b)

        def Gstart(c, b):
            base = sid * E_PER_TILE + c * EC
            pltpu.make_async_copy(src_hbm.at[pl.ds(base, EC)], sidx[b],
                                  isem[b]).wait()
            pltpu.make_async_copy(dst_hbm.at[pl.ds(base, EC)], didx[b],
                                  isem[b]).wait()
            pltpu.async_copy(xs_hbm.at[sidx[b]], rows[b], gsem[b])

        def S(c, b):
            pltpu.make_async_copy(xs_hbm.at[sidx[b]], rows[b], gsem[b]).wait()
            pltpu.sync_copy(rows[b], acc_sh.at[didx[b]], add=True)

        I(0, 0)
        Gstart(0, 0)
        I(1, 1)

        def body(p, carry):
            c0 = 2 * p
            c1 = c0 + 1
            Gstart(c1, 1)
            S(c0, 0)
            Iw(c0 + 2, 0)
            Gstart(c0 + 2, 0)
            S(c1, 1)
            Iw(c1 + 2, 1)
            return carry

        lax.fori_loop(0, (E_CHUNKS - 1) // 2, body, 0)
        S(E_CHUNKS - 1, 0)

    @pl.when(cid == 0)
    def _():
        run(xsA_hbm)

    @pl.when(cid == 1)
    def _():
        run(xsB_hbm)

    plsc.subcore_barrier()
    sl = pl.ds(sid * NSLICE, NSLICE)

    @pl.when(cid == 0)
    def _():
        pltpu.sync_copy(acc_sh.at[sl], outA_hbm.at[sl])

    @pl.when(cid == 1)
    def _():
        pltpu.sync_copy(acc_sh.at[sl], outB_hbm.at[sl])


@functools.partial(
    pl.kernel,
    out_type=(jax.ShapeDtypeStruct((G, HH), F32),
              jax.ShapeDtypeStruct((G, HH), F32),
              jax.ShapeDtypeStruct((G,), F32)),
    mesh=_mesh,
    compiler_params=pltpu.CompilerParams(use_tc_tiling_on_sc=False),
    scratch_types=[
        pltpu.VMEM((PC,), I32),
        pltpu.VMEM((PC, HH), F32),
        pltpu.VMEM((PC,), F32),
        pltpu.VMEM_SHARED((GP, HH), F32),
        pltpu.VMEM_SHARED((GP,), F32),
    ],
)
def _pool_kernel(hA_hbm, hB_hbm, batch_hbm, zeros2_hbm, zeros1_hbm, ones_hbm,
                 sumsA_hbm, sumsB_hbm, cnt_hbm,
                 bidx_v, rows_v, ones_v, accP_sh, accC_sh):
    cid = lax.axis_index("c")
    sid = lax.axis_index("s")

    @pl.when(sid == 0)
    def _():
        pltpu.sync_copy(zeros2_hbm.at[pl.ds(0, GP)], accP_sh)
        pltpu.sync_copy(zeros1_hbm.at[pl.ds(0, GP)], accC_sh)

    pltpu.sync_copy(ones_hbm, ones_v)
    plsc.subcore_barrier()

    def run(h_hbm, do_cnt):
        def body(i, carry):
            r0 = sid * NSLICE + i * PC
            pltpu.sync_copy(batch_hbm.at[pl.ds(r0, PC)], bidx_v)
            pltpu.sync_copy(h_hbm.at[pl.ds(r0, PC)], rows_v)
            pltpu.sync_copy(rows_v, accP_sh.at[bidx_v], add=True)
            if do_cnt:
                pltpu.sync_copy(ones_v, accC_sh.at[bidx_v], add=True)
            return carry
        lax.fori_loop(0, P_CHUNKS, body, 0)

    @pl.when(cid == 0)
    def _():
        run(hA_hbm, True)

    @pl.when(cid == 1)
    def _():
        run(hB_hbm, False)

    plsc.subcore_barrier()

    @pl.when(sid == 0)
    def _():
        @pl.when(cid == 0)
        def _():
            pltpu.sync_copy(accP_sh.at[pl.ds(0, G)], sumsA_hbm)
            pltpu.sync_copy(accC_sh.at[pl.ds(0, G)], cnt_hbm)

        @pl.when(cid == 1)
        def _():
            pltpu.sync_copy(accP_sh.at[pl.ds(0, G)], sumsB_hbm)


RB = 512
N_BLOCKS = NPAD // RB


def _leaky_tc(v):
    return jnp.where(v >= 0, v, 0.01 * v)


def _t0_body(xT_ref, dA_ref, dB_ref, W_ref, xsA_ref, xsB_ref, dv_ref):
    deg = dA_ref[...] + dB_ref[...] + 1.0
    dv = lax.rsqrt(deg)
    xw = lax.dot_general(xT_ref[...], W_ref[...], (((0,), (0,)), ((), ())),
                         preferred_element_type=F32)
    xs = xw * dv
    xsA_ref[...] = xs[:, :HH]
    xsB_ref[...] = xs[:, HH:]
    dv_ref[...] = dv


_t0_call = pl.pallas_call(
    _t0_body,
    grid=(N_BLOCKS,),
    in_specs=[
        pl.BlockSpec((NODE_IN, RB), lambda i: (0, i)),
        pl.BlockSpec((RB, 1), lambda i: (i, 0)),
        pl.BlockSpec((RB, 1), lambda i: (i, 0)),
        pl.BlockSpec((NODE_IN, H), lambda i: (0, 0)),
    ],
    out_specs=[
        pl.BlockSpec((RB, HH), lambda i: (i, 0)),
        pl.BlockSpec((RB, HH), lambda i: (i, 0)),
        pl.BlockSpec((RB, 1), lambda i: (i, 0)),
    ],
    out_shape=[
        jax.ShapeDtypeStruct((NPAD, HH), F32),
        jax.ShapeDtypeStruct((NPAD, HH), F32),
        jax.ShapeDtypeStruct((NPAD, 1), F32),
    ],
)


def _t12_body(SA_ref, SB_ref, xA_ref, xB_ref, dv_ref, b_ref, W_ref,
              outA_ref, outB_ref):
    dv = dv_ref[...]
    b = b_ref[...]
    hA = _leaky_tc(dv * (SA_ref[...] + xA_ref[...]) + b[:, :HH])
    hB = _leaky_tc(dv * (SB_ref[...] + xB_ref[...]) + b[:, HH:])
    h = jnp.concatenate([hA, hB], axis=1)
    xw = jnp.dot(h, W_ref[...], preferred_element_type=F32)
    xs = xw * dv
    outA_ref[...] = xs[:, :HH]
    outB_ref[...] = xs[:, HH:]


_t12_call = pl.pallas_call(
    _t12_body,
    grid=(N_BLOCKS,),
    in_specs=[
        pl.BlockSpec((RB, HH), lambda i: (i, 0)),
        pl.BlockSpec((RB, HH), lambda i: (i, 0)),
        pl.BlockSpec((RB, HH), lambda i: (i, 0)),
        pl.BlockSpec((RB, HH), lambda i: (i, 0)),
        pl.BlockSpec((RB, 1), lambda i: (i, 0)),
        pl.BlockSpec((1, H), lambda i: (0, 0)),
        pl.BlockSpec((H, H), lambda i: (0, 0)),
    ],
    out_specs=[
        pl.BlockSpec((RB, HH), lambda i: (i, 0)),
        pl.BlockSpec((RB, HH), lambda i: (i, 0)),
    ],
    out_shape=[
        jax.ShapeDtypeStruct((NPAD, HH), F32),
        jax.ShapeDtypeStruct((NPAD, HH), F32),
    ],
)


def _t3_body(SA_ref, SB_ref, xA_ref, xB_ref, dv_ref, b_ref,
             hA_ref, hB_ref):
    dv = dv_ref[...]
    b = b_ref[...]
    hA_ref[...] = _leaky_tc(dv * (SA_ref[...] + xA_ref[...]) + b[:, :HH])
    hB_ref[...] = _leaky_tc(dv * (SB_ref[...] + xB_ref[...]) + b[:, HH:])


_t3_call = pl.pallas_call(
    _t3_body,
    grid=(N_BLOCKS,),
    in_specs=[
        pl.BlockSpec((RB, HH), lambda i: (i, 0)),
        pl.BlockSpec((RB, HH), lambda i: (i, 0)),
        pl.BlockSpec((RB, HH), lambda i: (i, 0)),
        pl.BlockSpec((RB, HH), lambda i: (i, 0)),
        pl.BlockSpec((RB, 1), lambda i: (i, 0)),
        pl.BlockSpec((1, H), lambda i: (0, 0)),
    ],
    out_specs=[
        pl.BlockSpec((RB, HH), lambda i: (i, 0)),
        pl.BlockSpec((RB, HH), lambda i: (i, 0)),
    ],
    out_shape=[
        jax.ShapeDtypeStruct((NPAD, HH), F32),
        jax.ShapeDtypeStruct((NPAD, HH), F32),
    ],
)


def _head_body(sA_ref, sB_ref, cnt_ref, Wf1_ref, bf1_ref, Wf2_ref, bf2_ref,
               out_ref):
    pooled = jnp.concatenate([sA_ref[...], sB_ref[...]], axis=1)
    pooled = pooled / jnp.maximum(cnt_ref[...], 1.0)
    z = _leaky_tc(
        jnp.dot(pooled, Wf1_ref[...], preferred_element_type=F32)
        + bf1_ref[...])
    out_ref[...] = (
        jnp.dot(z, Wf2_ref[...], preferred_element_type=F32) + bf2_ref[...])


_head_call = pl.pallas_call(
    _head_body,
    out_shape=jax.ShapeDtypeStruct((G, 1), F32),
)


def kernel(x, edge_index, batch, W0, b0, W1, b1, W2, b2, Wf1, bf1, Wf2, bf2):
    xT = x.T
    src = edge_index[0]
    dst = edge_index[1]
    batch_pad = jnp.concatenate(
        [batch, jnp.full((NPAD - N,), G, dtype=I32)])
    zeros2 = jnp.zeros((NSLICE, HH), dtype=F32)
    zeros1 = jnp.zeros((NSLICE,), dtype=F32)
    ones_d = jnp.ones((DEG_EC,), dtype=F32)
    ones_p = jnp.ones((PC,), dtype=F32)

    degA, degB = _deg_kernel(dst, zeros1, ones_d)
    xsA, xsB, dv = _t0_call(xT, degA.reshape(NPAD, 1),
                            degB.reshape(NPAD, 1), W0)
    SA, SB = _edge_kernel(src, dst, xsA, xsB, zeros2)
    xsA, xsB = _t12_call(SA, SB, xsA, xsB, dv, b0.reshape(1, H), W1)
    SA, SB = _edge_kernel(src, dst, xsA, xsB, zeros2)
    xsA, xsB = _t12_call(SA, SB, xsA, xsB, dv, b1.reshape(1, H), W2)
    SA, SB = _edge_kernel(src, dst, xsA, xsB, zeros2)
    hA, hB = _t3_call(SA, SB, xsA, xsB, dv, b2.reshape(1, H))
    sumsA, sumsB, cnt = _pool_kernel(hA, hB, batch_pad, zeros2, zeros1, ones_p)
    out = _head_call(sumsA, sumsB, cnt.reshape(G, 1),
                     Wf1, bf1.reshape(1, H), Wf2, bf2.reshape(1, 1))
    return out.reshape(G)


# speedup vs baseline: 39.0065x; 1.5938x over previous
import functools

import jax
import jax.numpy as jnp
from jax import lax
from jax.experimental import pallas as pl
from jax.experimental.pallas import tpu as pltpu
from jax.experimental.pallas import tpu_sc as plsc

F32 = jnp.float32
I32 = jnp.int32

N = 50000
E = 800000
NODE_IN = 163
H = 64
HH = H // 2
G = 512

NPAD = 50176
EC = 400
E_PER_TILE = E // 16
E_CHUNKS = E_PER_TILE // EC
W_PER_DEG = E // 32
DEG_EC = 1000
DEG_CHUNKS = W_PER_DEG // DEG_EC
NSLICE = NPAD // 16
PC = 784
P_CHUNKS = NSLICE // PC
GP = 520

_mesh = plsc.VectorSubcoreMesh(core_axis_name="c", subcore_axis_name="s")


@functools.partial(
    pl.kernel,
    out_type=(jax.ShapeDtypeStruct((NPAD, HH), F32),
              jax.ShapeDtypeStruct((NPAD, HH), F32)),
    mesh=_mesh,
    compiler_params=pltpu.CompilerParams(use_tc_tiling_on_sc=False,
                                         needs_layout_passes=False),
    scratch_types=[
        pltpu.VMEM((DEG_EC,), I32),
        pltpu.VMEM((DEG_EC,), F32),
        pltpu.VMEM((NSLICE,), F32),
        pltpu.VMEM((NSLICE, HH), F32),
        pltpu.VMEM_SHARED((NPAD,), F32),
    ],
)
def _deg_kernel(ei_hbm, zeros1_hbm, ones_hbm, degA_hbm, degB_hbm,
                didx_v, ones_v, dvals_v, rep_v, acc_sh):
    cid = lax.axis_index("c")
    sid = lax.axis_index("s")
    pltpu.sync_copy(zeros1_hbm, acc_sh.at[pl.ds(sid * NSLICE, NSLICE)])
    pltpu.sync_copy(ones_hbm, ones_v)
    plsc.subcore_barrier()

    w = cid * 16 + sid

    def body(i, carry):
        base = w * W_PER_DEG + i * DEG_EC
        pltpu.sync_copy(ei_hbm.at[1, pl.ds(base, DEG_EC)], didx_v)
        pltpu.sync_copy(ones_v, acc_sh.at[didx_v], add=True)
        return carry

    lax.fori_loop(0, DEG_CHUNKS, body, 0)
    plsc.subcore_barrier()

    sl = pl.ds(sid * NSLICE, NSLICE)
    pltpu.sync_copy(acc_sh.at[sl], dvals_v)
    lanes = lax.iota(I32, 16)

    def gbody(g, carry):
        vals = dvals_v[pl.ds(g * 16, 16)]
        rowi = lanes + g * 16
        for j in range(HH):
            plsc.store_scatter(rep_v, [rowi, jnp.full((16,), j, I32)], vals)
        return carry

    lax.fori_loop(0, NSLICE // 16, gbody, 0)

    @pl.when(cid == 0)
    def _():
        pltpu.sync_copy(rep_v, degA_hbm.at[sl])

    @pl.when(cid == 1)
    def _():
        pltpu.sync_copy(rep_v, degB_hbm.at[sl])


@functools.partial(
    pl.kernel,
    out_type=(jax.ShapeDtypeStruct((NPAD, HH), F32),
              jax.ShapeDtypeStruct((NPAD, HH), F32)),
    mesh=_mesh,
    compiler_params=pltpu.CompilerParams(use_tc_tiling_on_sc=False),
    scratch_types=[
        pltpu.VMEM((EC,), I32),
        pltpu.VMEM((EC,), I32),
        pltpu.VMEM((EC,), I32),
        pltpu.VMEM((EC,), I32),
        pltpu.VMEM((EC, HH), F32),
        pltpu.VMEM((EC, HH), F32),
        pltpu.VMEM_SHARED((NPAD, HH), F32),
        pltpu.SemaphoreType.DMA,
        pltpu.SemaphoreType.DMA,
        pltpu.SemaphoreType.DMA,
        pltpu.SemaphoreType.DMA,
    ],
)
def _edge_kernel(ei_hbm, xsA_hbm, xsB_hbm, zeros2_hbm,
                 outA_hbm, outB_hbm,
                 sidx0, didx0, sidx1, didx1, rows0, rows1, acc_sh,
                 isem0, isem1, gsem0, gsem1):
    cid = lax.axis_index("c")
    sid = lax.axis_index("s")
    pltpu.sync_copy(zeros2_hbm, acc_sh.at[pl.ds(sid * NSLICE, NSLICE)])
    plsc.subcore_barrier()

    sidx = (sidx0, sidx1)
    didx = (didx0, didx1)
    rows = (rows0, rows1)
    isem = (isem0, isem1)
    gsem = (gsem0, gsem1)

    def run(xs_hbm):
        def I(c, b):
            base = sid * E_PER_TILE + c * EC
            pltpu.async_copy(ei_hbm.at[0, pl.ds(base, EC)], sidx[b], isem[b])
            pltpu.async_copy(ei_hbm.at[1, pl.ds(base, EC)], didx[b], isem[b])

        def Iw(c, b):
            @pl.when(c < E_CHUNKS)
            def _():
                I(c, b)

        def Gstart(c, b):
            base = sid * E_PER_TILE + c * EC
            pltpu.make_async_copy(ei_hbm.at[0, pl.ds(base, EC)], sidx[b],
                                  isem[b]).wait()
            pltpu.make_async_copy(ei_hbm.at[1, pl.ds(base, EC)], didx[b],
                                  isem[b]).wait()
            pltpu.async_copy(xs_hbm.at[sidx[b]], rows[b], gsem[b])

        def S(c, b):
            pltpu.make_async_copy(xs_hbm.at[sidx[b]], rows[b],
                                  gsem[b]).wait()
            pltpu.sync_copy(rows[b], acc_sh.at[didx[b]], add=True)

        I(0, 0)
        Gstart(0, 0)
        I(1, 1)

        def body(p, carry):
            c0 = 2 * p
            c1 = c0 + 1
            Gstart(c1, 1)
            S(c0, 0)
            Iw(c0 + 2, 0)
            Gstart(c0 + 2, 0)
            S(c1, 1)
            Iw(c1 + 2, 1)
            return carry

        lax.fori_loop(0, (E_CHUNKS - 1) // 2, body, 0)
        S(E_CHUNKS - 1, 0)

    @pl.when(cid == 0)
    def _():
        run(xsA_hbm)

    @pl.when(cid == 1)
    def _():
        run(xsB_hbm)

    plsc.subcore_barrier()
    sl = pl.ds(sid * NSLICE, NSLICE)

    @pl.when(cid == 0)
    def _():
        pltpu.sync_copy(acc_sh.at[sl], outA_hbm.at[sl])

    @pl.when(cid == 1)
    def _():
        pltpu.sync_copy(acc_sh.at[sl], outB_hbm.at[sl])


@functools.partial(
    pl.kernel,
    out_type=(jax.ShapeDtypeStruct((G, HH), F32),
              jax.ShapeDtypeStruct((G, HH), F32),
              jax.ShapeDtypeStruct((G, HH), F32)),
    mesh=_mesh,
    compiler_params=pltpu.CompilerParams(use_tc_tiling_on_sc=False),
    scratch_types=[
        pltpu.VMEM((PC,), I32),
        pltpu.VMEM((PC, HH), F32),
        pltpu.VMEM((PC, HH), F32),
        pltpu.VMEM_SHARED((GP, HH), F32),
        pltpu.VMEM_SHARED((GP, HH), F32),
    ],
)
def _pool_kernel(hA_hbm, hB_hbm, batch_hbm, zeros2_hbm, ones_hbm,
                 sumsA_hbm, sumsB_hbm, cnt_hbm,
                 bidx_v, rows_v, ones_v, accP_sh, accC_sh):
    cid = lax.axis_index("c")
    sid = lax.axis_index("s")

    @pl.when(sid == 0)
    def _():
        pltpu.sync_copy(zeros2_hbm.at[pl.ds(0, GP)], accP_sh)
        pltpu.sync_copy(zeros2_hbm.at[pl.ds(0, GP)], accC_sh)

    pltpu.sync_copy(ones_hbm, ones_v)
    plsc.subcore_barrier()

    def run(h_hbm, do_cnt):
        def body(i, carry):
            r0 = sid * NSLICE + i * PC
            pltpu.sync_copy(batch_hbm.at[pl.ds(r0, PC)], bidx_v)
            pltpu.sync_copy(h_hbm.at[pl.ds(r0, PC)], rows_v)
            pltpu.sync_copy(rows_v, accP_sh.at[bidx_v], add=True)
            if do_cnt:
                pltpu.sync_copy(ones_v, accC_sh.at[bidx_v], add=True)
            return carry
        lax.fori_loop(0, P_CHUNKS, body, 0)

    @pl.when(cid == 0)
    def _():
        run(hA_hbm, True)

    @pl.when(cid == 1)
    def _():
        run(hB_hbm, False)

    plsc.subcore_barrier()

    @pl.when(sid == 0)
    def _():
        @pl.when(cid == 0)
        def _():
            pltpu.sync_copy(accP_sh.at[pl.ds(0, G)], sumsA_hbm)
            pltpu.sync_copy(accC_sh.at[pl.ds(0, G)], cnt_hbm)

        @pl.when(cid == 1)
        def _():
            pltpu.sync_copy(accP_sh.at[pl.ds(0, G)], sumsB_hbm)


RB = 1024
N_BLOCKS = NPAD // RB
PK = RB // 4
NP4 = NPAD // 4


def _leaky_tc(v):
    return jnp.where(v >= 0, v, 0.01 * v)


def _t0_body(xT_ref, W_ref, xwA_ref, xwB_ref):
    xw = lax.dot_general(xT_ref[...], W_ref[...], (((0,), (0,)), ((), ())),
                         preferred_element_type=F32)
    xwA_ref[...] = xw[:, :HH]
    xwB_ref[...] = xw[:, HH:]


_t0_call = pl.pallas_call(
    _t0_body,
    grid=(N_BLOCKS,),
    in_specs=[
        pl.BlockSpec((NODE_IN, RB), lambda i: (0, i)),
        pl.BlockSpec((NODE_IN, H), lambda i: (0, 0)),
    ],
    out_specs=[
        pl.BlockSpec((RB, HH), lambda i: (i, 0)),
        pl.BlockSpec((RB, HH), lambda i: (i, 0)),
    ],
    out_shape=[
        jax.ShapeDtypeStruct((NPAD, HH), F32),
        jax.ShapeDtypeStruct((NPAD, HH), F32),
    ],
)


def _t0b_body(xwA_ref, xwB_ref, dfA_ref, dfB_ref, xsA_ref, xsB_ref, dv_ref):
    dv4 = lax.rsqrt(dfA_ref[...] + dfB_ref[...] + 1.0)
    xsA_ref[...] = xwA_ref[...] * dv4
    xsB_ref[...] = xwB_ref[...] * dv4
    dv_ref[...] = dv4


_t0b_call = pl.pallas_call(
    _t0b_body,
    grid=(N_BLOCKS,),
    in_specs=[pl.BlockSpec((PK, 128), lambda i: (i, 0))] * 4,
    out_specs=[pl.BlockSpec((PK, 128), lambda i: (i, 0))] * 3,
    out_shape=[jax.ShapeDtypeStruct((NP4, 128), F32)] * 3,
)


def _t12_body(SA_ref, SB_ref, xA_ref, xB_ref, dv_ref,
              bA_ref, bB_ref, WAA_ref, WBA_ref, WAB_ref, WBB_ref,
              outA_ref, outB_ref):
    dv4 = dv_ref[...]
    hA = _leaky_tc(dv4 * (SA_ref[...] + xA_ref[...]) + bA_ref[...])
    hB = _leaky_tc(dv4 * (SB_ref[...] + xB_ref[...]) + bB_ref[...])
    xwA = (jnp.dot(hA, WAA_ref[...], preferred_element_type=F32)
           + jnp.dot(hB, WBA_ref[...], preferred_element_type=F32))
    xwB = (jnp.dot(hA, WAB_ref[...], preferred_element_type=F32)
           + jnp.dot(hB, WBB_ref[...], preferred_element_type=F32))
    outA_ref[...] = xwA * dv4
    outB_ref[...] = xwB * dv4


_t12_call = pl.pallas_call(
    _t12_body,
    grid=(N_BLOCKS,),
    in_specs=[pl.BlockSpec((PK, 128), lambda i: (i, 0))] * 5 + [
        pl.BlockSpec((1, 128), lambda i: (0, 0)),
        pl.BlockSpec((1, 128), lambda i: (0, 0)),
        pl.BlockSpec((128, 128), lambda i: (0, 0)),
        pl.BlockSpec((128, 128), lambda i: (0, 0)),
        pl.BlockSpec((128, 128), lambda i: (0, 0)),
        pl.BlockSpec((128, 128), lambda i: (0, 0)),
    ],
    out_specs=[pl.BlockSpec((PK, 128), lambda i: (i, 0))] * 2,
    out_shape=[jax.ShapeDtypeStruct((NP4, 128), F32)] * 2,
)


def _t3_body(SA_ref, SB_ref, xA_ref, xB_ref, dv_ref, bA_ref, bB_ref,
             hA_ref, hB_ref):
    dv4 = dv_ref[...]
    hA_ref[...] = _leaky_tc(dv4 * (SA_ref[...] + xA_ref[...]) + bA_ref[...])
    hB_ref[...] = _leaky_tc(dv4 * (SB_ref[...] + xB_ref[...]) + bB_ref[...])


_t3_call = pl.pallas_call(
    _t3_body,
    grid=(N_BLOCKS,),
    in_specs=[pl.BlockSpec((PK, 128), lambda i: (i, 0))] * 5 + [
        pl.BlockSpec((1, 128), lambda i: (0, 0)),
        pl.BlockSpec((1, 128), lambda i: (0, 0)),
    ],
    out_specs=[pl.BlockSpec((PK, 128), lambda i: (i, 0))] * 2,
    out_shape=[jax.ShapeDtypeStruct((NP4, 128), F32)] * 2,
)


def _head_body(sA_ref, sB_ref, cnt_ref,
               FAA_ref, FBA_ref, FAB_ref, FBB_ref, bfA_ref, bfB_ref,
               W2A_ref, W2B_ref, bf2_ref, out_ref):
    c = jnp.maximum(cnt_ref[...], 1.0)
    pA = sA_ref[...] / c
    pB = sB_ref[...] / c
    zA = _leaky_tc(jnp.dot(pA, FAA_ref[...], preferred_element_type=F32)
                   + jnp.dot(pB, FBA_ref[...], preferred_element_type=F32)
                   + bfA_ref[...])
    zB = _leaky_tc(jnp.dot(pA, FAB_ref[...], preferred_element_type=F32)
                   + jnp.dot(pB, FBB_ref[...], preferred_element_type=F32)
                   + bfB_ref[...])
    out_ref[...] = (jnp.dot(zA, W2A_ref[...], preferred_element_type=F32)
                    + jnp.dot(zB, W2B_ref[...], preferred_element_type=F32)
                    + bf2_ref[...])


_head_call = pl.pallas_call(
    _head_body,
    out_shape=jax.ShapeDtypeStruct((G // 4, 4), F32),
)


def kernel(x, edge_index, batch, W0, b0, W1, b1, W2, b2, Wf1, bf1, Wf2, bf2):
    xT = x.T
    batch_pad = jnp.concatenate(
        [batch, jnp.full((NPAD - N,), G, dtype=I32)])
    zeros2 = jnp.zeros((NSLICE, HH), dtype=F32)
    zeros1 = jnp.zeros((NSLICE,), dtype=F32)
    ones_d = jnp.ones((DEG_EC,), dtype=F32)
    ones_p = jnp.ones((PC, HH), dtype=F32)

    def pack(a):
        return a.reshape(NP4, 128)

    def unpack(a):
        return a.reshape(NPAD, HH)

    eye4 = jnp.eye(4, dtype=F32)

    def bd(M):
        return jnp.kron(eye4, M)

    def tile4(v):
        return jnp.tile(v, 4).reshape(1, 128)

    degAf, degBf = _deg_kernel(edge_index, zeros1, ones_d)
    xwA, xwB = _t0_call(xT, W0)
    xsA_p, xsB_p, dv_p = _t0b_call(pack(xwA), pack(xwB),
                                   pack(degAf), pack(degBf))
    SA, SB = _edge_kernel(edge_index, unpack(xsA_p), unpack(xsB_p), zeros2)
    xsA_p, xsB_p = _t12_call(
        pack(SA), pack(SB), xsA_p, xsB_p, dv_p,
        tile4(b0[:HH]), tile4(b0[HH:]),
        bd(W1[:HH, :HH]), bd(W1[HH:, :HH]), bd(W1[:HH, HH:]), bd(W1[HH:, HH:]))
    SA, SB = _edge_kernel(edge_index, unpack(xsA_p), unpack(xsB_p), zeros2)
    xsA_p, xsB_p = _t12_call(
        pack(SA), pack(SB), xsA_p, xsB_p, dv_p,
        tile4(b1[:HH]), tile4(b1[HH:]),
        bd(W2[:HH, :HH]), bd(W2[HH:, :HH]), bd(W2[:HH, HH:]), bd(W2[HH:, HH:]))
    SA, SB = _edge_kernel(edge_index, unpack(xsA_p), unpack(xsB_p), zeros2)
    hA_p, hB_p = _t3_call(pack(SA), pack(SB), xsA_p, xsB_p, dv_p,
                          tile4(b2[:HH]), tile4(b2[HH:]))
    sumsA, sumsB, cntf = _pool_kernel(unpack(hA_p), unpack(hB_p), batch_pad,
                                      zeros2, ones_p)
    out4 = _head_call(
        sumsA.reshape(G // 4, 128), sumsB.reshape(G // 4, 128),
        cntf.reshape(G // 4, 128),
        bd(Wf1[:HH, :HH]), bd(Wf1[HH:, :HH]), bd(Wf1[:HH, HH:]),
        bd(Wf1[HH:, HH:]),
        tile4(bf1[:HH]), tile4(bf1[HH:]),
        bd(Wf2[:HH, :]), bd(Wf2[HH:, :]),
        bf2.reshape(1, 1))
    return out4.reshape(G)
